# Initial kernel scaffold; baseline (speedup 1.0000x reference)
import jax
import jax.numpy as jnp
from jax.experimental import pallas as pl


def kernel(x, edge_index, edge_attr, W1_rel, W1_root, b1, W2_rel, W2_root, b2, Wfc, bfc, Wlast, blast):
    raise NotImplementedError("write your pallas kernel here")


# speedup vs baseline: 2.9259x; 2.9259x over previous
import functools

import jax
import jax.numpy as jnp
from jax import lax
from jax.experimental import pallas as pl
from jax.experimental.pallas import tpu as pltpu
from jax.experimental.pallas import tpu_sc as plsc

N = 10000
NP = 10240
E = 320000
NC = 2
NS = 16
NW = NC * NS
EPW = E // NW
CH = 80
NCHUNK = EPW // CH
RPS = NP // NS
ZROWS = 128
F32 = jnp.float32


def _splat(v16, j):
    idx = jnp.full((16, 1), j, jnp.int32)
    dn = lax.GatherDimensionNumbers(
        offset_dims=(), collapsed_slice_dims=(0,), start_index_map=(0,))
    return lax.gather(v16, idx, dn, slice_sizes=(1,),
                      mode=lax.GatherScatterMode.PROMISE_IN_BOUNDS)


def _segsum(parts, src, dst, w):
    nparts = len(parts)
    mesh = plsc.VectorSubcoreMesh(core_axis_name="c", subcore_axis_name="s")
    out_type = jax.ShapeDtypeStruct((NC, nparts, NP, 128), F32)
    scratch = [
        pltpu.VMEM((CH,), jnp.int32),
        pltpu.VMEM((CH,), jnp.int32),
        pltpu.VMEM((CH,), F32),
        pltpu.VMEM((CH, 128), F32),
        pltpu.VMEM((ZROWS, 128), F32),
        pltpu.VMEM_SHARED((NP, 128), F32),
        pltpu.SemaphoreType.DMA,
    ]

    @functools.partial(pl.kernel, out_type=out_type, mesh=mesh,
                       scratch_types=scratch)
    def k(*refs):
        part_h = refs[:nparts]
        (src_h, dst_h, w_h, out_h,
         src_v, dst_v, w_v, rows_v, zbuf, acc, sem) = refs[nparts:]
        c = lax.axis_index("c")
        s = lax.axis_index("s")
        base0 = (s * NC + c) * EPW

        zero = jnp.zeros((16,), F32)

        @pl.loop(0, ZROWS)
        def _(r):
            for cc in range(8):
                zbuf.at[r, pl.ds(cc * 16, 16)][...] = zero

        for p in range(nparts):
            for blk in range(RPS // ZROWS):
                pltpu.sync_copy(zbuf, acc.at[pl.ds(s * RPS + blk * ZROWS,
                                                   ZROWS)])
            plsc.subcore_barrier()

            @pl.loop(0, NCHUNK)
            def _(kk):
                base = base0 + kk * CH
                pltpu.sync_copy(src_h.at[pl.ds(base, CH)], src_v)
                pltpu.sync_copy(dst_h.at[pl.ds(base, CH)], dst_v)
                pltpu.sync_copy(w_h.at[pl.ds(base, CH)], w_v)
                pltpu.async_copy(part_h[p].at[src_v], rows_v, sem).wait()

                @pl.loop(0, CH // 16)
                def _(g):
                    w16 = w_v[pl.ds(g * 16, 16)]
                    for j in range(16):
                        wj = _splat(w16, j)
                        for cc in range(8):
                            sl = (g * 16 + j, pl.ds(cc * 16, 16))
                            rows_v.at[sl][...] = rows_v.at[sl][...] * wj

                pltpu.sync_copy(rows_v, acc.at[dst_v], add=True)

            plsc.subcore_barrier()
            pltpu.sync_copy(acc.at[pl.ds(s * RPS, RPS)],
                            out_h.at[c, p, pl.ds(s * RPS, RPS)])
            plsc.subcore_barrier()

    return k(*parts, src, dst, w)


def _gnn_layer(P, hs, W_rel, W_root, b):
    nparts = P.shape[1]
    BN = 1000
    grid = (N // BN,)
    in_specs = [pl.BlockSpec((NC, nparts, BN, 128), lambda i: (0, 0, i, 0))]
    in_specs += [pl.BlockSpec((BN, 128), lambda i: (i, 0)) for _ in hs]
    in_specs += [
        pl.BlockSpec(W_rel.shape, lambda i: (0, 0)),
        pl.BlockSpec(W_root.shape, lambda i: (0, 0)),
        pl.BlockSpec((1, 256), lambda i: (0, 0)),
    ]
    out_specs = [pl.BlockSpec((BN, 128), lambda i: (i, 0))] * 2
    nh = len(hs)

    def body(P_ref, *refs):
        h_refs = refs[:nh]
        wrel, wroot, b_ref, olo, ohi = refs[nh:]
        acc = jnp.zeros((BN, 256), F32)
        for p in range(nparts):
            aggp = P_ref[0, p] + P_ref[1, p]
            acc += jnp.dot(aggp, wrel[p * 128:(p + 1) * 128],
                           preferred_element_type=F32)
        for q in range(nh):
            acc += jnp.dot(h_refs[q][...], wroot[q * 128:(q + 1) * 128],
                           preferred_element_type=F32)
        z = jnp.maximum(acc + b_ref[...], 0.0)
        olo[...] = z[:, :128]
        ohi[...] = z[:, 128:]

    return pl.pallas_call(
        body, grid=grid, in_specs=in_specs, out_specs=out_specs,
        out_shape=[jax.ShapeDtypeStruct((N, 128), F32)] * 2,
    )(P, *hs, W_rel, W_root, b.reshape(1, -1))


def _mlp_head(h_lo, h_hi, Wfc, bfc, Wlast, blast):
    BN = 1000
    grid = (N // BN,)
    in_specs = [
        pl.BlockSpec((BN, 128), lambda i: (i, 0)),
        pl.BlockSpec((BN, 128), lambda i: (i, 0)),
        pl.BlockSpec(Wfc.shape, lambda i: (0, 0)),
        pl.BlockSpec((1, 256), lambda i: (0, 0)),
        pl.BlockSpec(Wlast.shape, lambda i: (0, 0)),
        pl.BlockSpec((1, Wlast.shape[1]), lambda i: (0, 0)),
    ]
    out_specs = pl.BlockSpec((BN, Wlast.shape[1]), lambda i: (i, 0))

    def body(hlo, hhi, wfc, bfc_r, wlast, blast_r, o):
        t = (jnp.dot(hlo[...], wfc[:128], preferred_element_type=F32)
             + jnp.dot(hhi[...], wfc[128:], preferred_element_type=F32)
             + bfc_r[...])
        t = jnp.maximum(t, 0.0)
        t = jnp.maximum(
            jnp.dot(t, wfc[...], preferred_element_type=F32) + bfc_r[...],
            0.0)
        o[...] = jnp.dot(t, wlast[...], preferred_element_type=F32) \
            + blast_r[...]

    return pl.pallas_call(
        body, grid=grid, in_specs=in_specs, out_specs=out_specs,
        out_shape=jax.ShapeDtypeStruct((N, Wlast.shape[1]), F32),
    )(h_lo, h_hi, Wfc, bfc.reshape(1, -1), Wlast, blast.reshape(1, -1))


def kernel(x, edge_index, edge_attr, W1_rel, W1_root, b1,
           W2_rel, W2_root, b2, Wfc, bfc, Wlast, blast):
    src = edge_index[0]
    dst = edge_index[1]

    P1 = _segsum([x], src, dst, edge_attr)
    h1_lo, h1_hi = _gnn_layer(P1, [x], W1_rel, W1_root, b1)

    P2 = _segsum([h1_lo, h1_hi], src, dst, edge_attr)
    h2_lo, h2_hi = _gnn_layer(P2, [h1_lo, h1_hi], W2_rel, W2_root, b2)

    P3 = _segsum([h2_lo, h2_hi], src, dst, edge_attr)
    h3_lo, h3_hi = _gnn_layer(P3, [h2_lo, h2_hi], W2_rel, W2_root, b2)

    return _mlp_head(h3_lo, h3_hi, Wfc, bfc, Wlast, blast)
